# baseline (speedup 1.0000x reference)
import functools

import jax
import jax.numpy as jnp
from jax.experimental import pallas as pl
from jax.experimental.pallas import tpu as pltpu

H1, C1 = 4, 128
H2, C2 = 4, 256


def _mm_kern(a_ref, b_ref, o_ref):
    o_ref[...] = jnp.dot(a_ref[...], b_ref[...],
                         preferred_element_type=jnp.float32)


def _pmm(a, b, bm=512):
    M, K = a.shape
    _, N = b.shape
    pad = (-M) % bm
    if pad:
        a = jnp.pad(a, ((0, pad), (0, 0)))
    Mp = M + pad
    out = pl.pallas_call(
        _mm_kern,
        grid=(Mp // bm,),
        in_specs=[pl.BlockSpec((bm, K), lambda i: (i, 0)),
                  pl.BlockSpec((K, N), lambda i: (0, 0))],
        out_specs=pl.BlockSpec((bm, N), lambda i: (i, 0)),
        out_shape=jax.ShapeDtypeStruct((Mp, N), jnp.float32),
    )(a, b)
    return out[:M] if pad else out


def _layer1(xa, src, dst, ea, Wqa, Wka, Wva, We, Wsa):
    N = xa.shape[0]
    D = xa.shape[1]
    Wqa_h = Wqa.reshape(D, H1, C1)
    Wka_h = Wka.reshape(D, H1, C1)
    We_h = We.reshape(4, H1, C1)
    M1 = jnp.einsum('dhc,ehc->hde', Wqa_h, Wka_h)
    B1 = jnp.einsum('dhc,ehc->hde', Wqa_h, We_h)
    zk = jnp.einsum('ne,hde->nhd', xa, M1)
    ze = jnp.einsum('nd,hde->nhe', xa, B1)
    logits = (jnp.einsum('ed,ehd->eh', xa[dst], zk[src])
              + jnp.einsum('ehd,ed->eh', ze[dst], ea)) / jnp.sqrt(float(C1))
    p = jnp.exp(logits)
    ssum = jax.ops.segment_sum(p, dst, num_segments=N)
    alpha = p / (ssum[dst] + 1e-16)
    P = jax.ops.segment_sum(alpha[:, :, None] * xa[src][:, None, :], dst,
                            num_segments=N)
    T = jax.ops.segment_sum(alpha[:, :, None] * ea[:, None, :], dst,
                            num_segments=N)
    out = (jnp.einsum('nhd,dhc->nhc', P, Wva.reshape(D, H1, C1))
           + jnp.einsum('nhd,dhc->nhc', T, We_h)).reshape(N, H1 * C1)
    return out + xa @ Wsa


def _layer2(ha, src, dst, ea, Wqa, Wka, Wva, We, Wsa):
    N = ha.shape[0]
    qkvs = _pmm(ha, jnp.concatenate([Wqa, Wka, Wva, Wsa], axis=1))
    HC = H2 * C2
    q = qkvs[:, :HC].reshape(N, H2, C2)
    k = qkvs[:, HC:2 * HC].reshape(N, H2, C2)
    v = qkvs[:, 2 * HC:3 * HC].reshape(N, H2, C2)
    s = qkvs[:, 3 * HC:]
    We_h = We.reshape(4, H2, C2)
    ze = jnp.einsum('nhc,dhc->nhd', q, We_h)
    logits = (jnp.einsum('ehc,ehc->eh', q[dst], k[src])
              + jnp.einsum('ehd,ed->eh', ze[dst], ea)) / jnp.sqrt(float(C2))
    p = jnp.exp(logits)
    ssum = jax.ops.segment_sum(p, dst, num_segments=N)
    alpha = p / (ssum[dst] + 1e-16)
    U = jax.ops.segment_sum(alpha[:, :, None] * v[src], dst,
                            num_segments=N)
    T = jax.ops.segment_sum(alpha[:, :, None] * ea[:, None, :], dst,
                            num_segments=N)
    out = (U + jnp.einsum('nhd,dhc->nhc', T, We_h)).reshape(N, HC)
    return out + s


def kernel(x, edge_index, edge_attr, batch, fp_batch,
           Wq1, bq1, Wk1, bk1, Wv1, bv1, We1, Ws1, bs1,
           Wq2, bq2, Wk2, bk2, Wv2, bv2, We2, Ws2, bs2,
           Wfp, bfp, Wf, bf):
    src, dst = edge_index[0], edge_index[1]
    N = x.shape[0]
    xa = jnp.concatenate([x, jnp.ones((N, 1), jnp.float32)], axis=1)

    def aug(W, b):
        return jnp.concatenate([W, b[None, :]], axis=0)

    h1 = _layer1(xa, src, dst, edge_attr,
                 aug(Wq1, bq1), aug(Wk1, bk1), aug(Wv1, bv1), We1,
                 aug(Ws1, bs1))
    h1 = jax.nn.relu(h1)
    h1a = jnp.concatenate([h1, jnp.ones((N, 1), jnp.float32)], axis=1)
    h2 = _layer2(h1a, src, dst, edge_attr,
                 aug(Wq2, bq2), aug(Wk2, bk2), aug(Wv2, bv2), We2,
                 aug(Ws2, bs2))
    B = fp_batch.shape[0]
    ssum = jax.ops.segment_sum(h2, batch, num_segments=B)
    cnt = jax.ops.segment_sum(jnp.ones((N,), jnp.float32), batch,
                              num_segments=B)
    g = ssum / jnp.maximum(cnt, 1.0)[:, None]
    fp = _pmm(fp_batch, Wfp, bm=1024) + bfp
    return _pmm(jnp.concatenate([g, fp], axis=-1), Wf, bm=1024) + bf


# speedup vs baseline: 5.6238x; 5.6238x over previous
import functools

import jax
import jax.numpy as jnp
from jax import lax
from jax.experimental import pallas as pl
from jax.experimental.pallas import tpu as pltpu
from jax.experimental.pallas import tpu_sc as plsc

H1, C1 = 4, 128
H2, C2 = 4, 256

_NC, _NS = 2, 16
_NW = _NC * _NS
_CB = 256
_WS = 8


def _sc_segsum(vals, idx, n_out):
    E, D = vals.shape
    assert E % (_NW * 16) == 0
    epw = E // _NW
    ntp = -(-n_out // 128) * 128
    valsT = vals.T
    zeros = jnp.zeros((ntp,), jnp.float32)
    mesh = plsc.VectorSubcoreMesh(core_axis_name="c", subcore_axis_name="s")

    @functools.partial(
        pl.kernel, mesh=mesh,
        out_type=jax.ShapeDtypeStruct((_NW, D, ntp), jnp.float32),
        compiler_params=pltpu.CompilerParams(use_tc_tiling_on_sc=False),
        scratch_types=[
            pltpu.VMEM((epw,), jnp.int32),
            pltpu.VMEM((epw,), jnp.float32),
            pltpu.VMEM((ntp,), jnp.float32),
        ],
    )
    def k(valsT_hbm, idx_hbm, zeros_hbm, out_hbm, ibuf, cbuf, table):
        c = lax.axis_index("c")
        sid = lax.axis_index("s")
        w = c * _NS + sid
        ebase = w * epw
        pltpu.sync_copy(idx_hbm.at[pl.ds(ebase, epw)], ibuf)

        def col_body(d, _):
            pltpu.sync_copy(zeros_hbm, table)
            pltpu.sync_copy(valsT_hbm.at[d, pl.ds(ebase, epw)], cbuf)

            def grp(g, _):
                i16 = ibuf[pl.ds(g * 16, 16)]
                x16 = cbuf[pl.ds(g * 16, 16)]
                plsc.addupdate_scatter(table, [i16], x16)
                return 0

            lax.fori_loop(0, epw // 16, grp, 0)
            pltpu.sync_copy(table, out_hbm.at[w, d])
            return 0

        lax.fori_loop(0, D, col_body, 0)

    out = k(valsT, idx, zeros)
    return out.sum(axis=0).T[:n_out]


def _pad_rows(a, ep):
    E = a.shape[0]
    return a if E == ep else jnp.pad(a, ((0, ep - E),) + ((0, 0),) * (a.ndim - 1))


def _mm_kern(a_ref, b_ref, o_ref):
    o_ref[...] = jnp.dot(a_ref[...], b_ref[...],
                         preferred_element_type=jnp.float32)


def _pmm(a, b, bm=512):
    M, K = a.shape
    _, N = b.shape
    pad = (-M) % bm
    if pad:
        a = jnp.pad(a, ((0, pad), (0, 0)))
    Mp = M + pad
    out = pl.pallas_call(
        _mm_kern,
        grid=(Mp // bm,),
        in_specs=[pl.BlockSpec((bm, K), lambda i: (i, 0)),
                  pl.BlockSpec((K, N), lambda i: (0, 0))],
        out_specs=pl.BlockSpec((bm, N), lambda i: (i, 0)),
        out_shape=jax.ShapeDtypeStruct((Mp, N), jnp.float32),
    )(a, b)
    return out[:M] if pad else out


def _layer1(xa, src, dst, ea, emask, Wqa, Wka, Wva, We, Wsa):
    N = xa.shape[0]
    D = xa.shape[1]
    Wqa_h = Wqa.reshape(D, H1, C1)
    Wka_h = Wka.reshape(D, H1, C1)
    We_h = We.reshape(4, H1, C1)
    M1 = jnp.einsum('dhc,ehc->hde', Wqa_h, Wka_h)
    B1 = jnp.einsum('dhc,ehc->hde', Wqa_h, We_h)
    zk = jnp.einsum('ne,hde->nhd', xa, M1)
    ze = jnp.einsum('nd,hde->nhe', xa, B1)
    logits = (jnp.einsum('ed,ehd->eh', xa[dst], zk[src])
              + jnp.einsum('ehd,ed->eh', ze[dst], ea)) / jnp.sqrt(float(C1))
    p = jnp.exp(logits) * emask[:, None]
    ssum = jax.ops.segment_sum(p, dst, num_segments=N,
                               indices_are_sorted=True)
    alpha = p / (ssum[dst] + 1e-16)
    pt = jnp.concatenate(
        [(alpha[:, :, None] * xa[src][:, None, :]).reshape(-1, H1 * D),
         (alpha[:, :, None] * ea[:, None, :]).reshape(-1, H1 * 4),
         jnp.zeros((alpha.shape[0], 8), jnp.float32)], axis=1)
    PT = jax.ops.segment_sum(pt, dst, num_segments=N,
                             indices_are_sorted=True)
    P = PT[:, :H1 * D].reshape(N, H1, D)
    T = PT[:, H1 * D:H1 * D + H1 * 4].reshape(N, H1, 4)
    out = (jnp.einsum('nhd,dhc->nhc', P, Wva.reshape(D, H1, C1))
           + jnp.einsum('nhd,dhc->nhc', T, We_h)).reshape(N, H1 * C1)
    return out + xa @ Wsa


def _layer2(ha, src, dst, ea, emask, Wqa, Wka, Wva, We, Wsa):
    N = ha.shape[0]
    qkvs = _pmm(ha, jnp.concatenate([Wqa, Wka, Wva, Wsa], axis=1))
    HC = H2 * C2
    q = qkvs[:, :HC].reshape(N, H2, C2)
    k = qkvs[:, HC:2 * HC].reshape(N, H2, C2)
    v = qkvs[:, 2 * HC:3 * HC].reshape(N, H2, C2)
    s = qkvs[:, 3 * HC:]
    We_h = We.reshape(4, H2, C2)
    ze = jnp.einsum('nhc,dhc->nhd', q, We_h)
    logits = (jnp.einsum('ehc,ehc->eh', q[dst], k[src])
              + jnp.einsum('ehd,ed->eh', ze[dst], ea)) / jnp.sqrt(float(C2))
    p = jnp.exp(logits) * emask[:, None]
    ssum = jax.ops.segment_sum(p, dst, num_segments=N,
                               indices_are_sorted=True)
    alpha = p / (ssum[dst] + 1e-16)
    msg = jnp.concatenate(
        [(alpha[:, :, None] * v[src]).reshape(-1, HC),
         (alpha[:, :, None] * ea[:, None, :]).reshape(-1, H2 * 4)], axis=1)
    UT = jax.ops.segment_sum(msg, dst, num_segments=N,
                             indices_are_sorted=True)
    U = UT[:, :HC].reshape(N, H2, C2)
    T = UT[:, HC:].reshape(N, H2, 4)
    out = (U + jnp.einsum('nhd,dhc->nhc', T, We_h)).reshape(N, HC)
    return out + s


def kernel(x, edge_index, edge_attr, batch, fp_batch,
           Wq1, bq1, Wk1, bk1, Wv1, bv1, We1, Ws1, bs1,
           Wq2, bq2, Wk2, bk2, Wv2, bv2, We2, Ws2, bs2,
           Wfp, bfp, Wf, bf):
    N = x.shape[0]
    E = edge_index.shape[1]
    dst0 = edge_index[1].astype(jnp.int32)
    order = jnp.argsort(dst0)
    src = edge_index[0].astype(jnp.int32)[order]
    dst = dst0[order]
    ea = edge_attr[order]
    emask = jnp.ones((E,), jnp.float32)
    xa = jnp.concatenate([x, jnp.ones((N, 1), jnp.float32)], axis=1)

    def aug(W, b):
        return jnp.concatenate([W, b[None, :]], axis=0)

    h1 = _layer1(xa, src, dst, ea, emask,
                 aug(Wq1, bq1), aug(Wk1, bk1), aug(Wv1, bv1), We1,
                 aug(Ws1, bs1))
    h1 = jax.nn.relu(h1)
    h1a = jnp.concatenate([h1, jnp.ones((N, 1), jnp.float32)], axis=1)
    h2 = _layer2(h1a, src, dst, ea, emask,
                 aug(Wq2, bq2), aug(Wk2, bk2), aug(Wv2, bv2), We2,
                 aug(Ws2, bs2))
    B = fp_batch.shape[0]
    h2c = jnp.concatenate([h2, jnp.ones((N, 1), jnp.float32)], axis=1)
    sc = jax.ops.segment_sum(h2c, batch, num_segments=B,
                             indices_are_sorted=True)
    g = sc[:, :-1] / jnp.maximum(sc[:, -1:], 1.0)
    fp = _pmm(fp_batch, Wfp, bm=1024) + bfp
    return _pmm(jnp.concatenate([g, fp], axis=-1), Wf, bm=1024) + bf
